# Initial kernel scaffold; baseline (speedup 1.0000x reference)
import jax
import jax.numpy as jnp
from jax.experimental import pallas as pl


def kernel(input_ids, segment_ids, word_emb, W2, pos_emb, type_emb, gamma, beta):
    raise NotImplementedError("write your pallas kernel here")


# speedup vs baseline: 2.4336x; 2.4336x over previous
import functools

import jax
import jax.numpy as jnp
from jax import lax
from jax.experimental import pallas as pl
from jax.experimental.pallas import tpu as pltpu


def _dense_body(g_ref, s_ref, w_ref, p_ref, t_ref, gm_ref, bt_ref, o_ref):
    bb, L, wd = g_ref.shape
    dim = w_ref.shape[1]
    g = g_ref[...].reshape(bb * L, wd)
    x = jnp.dot(g, w_ref[...], preferred_element_type=jnp.float32)
    x = x.reshape(bb, L, dim)
    x = x + p_ref[...][None, :, :]
    s = s_ref[...]
    t = t_ref[...]
    m1 = lax.broadcast_in_dim((s == 1).astype(jnp.float32), (bb, L, dim), (0, 1))
    m2 = lax.broadcast_in_dim((s == 2).astype(jnp.float32), (bb, L, dim), (0, 1))
    t0 = t[0][None, None, :]
    t1 = t[1][None, None, :]
    t2 = t[2][None, None, :]
    x = x + t0 + m1 * (t1 - t0) + m2 * (t2 - t0)
    mean = jnp.mean(x, axis=-1, keepdims=True)
    xc = x - mean
    var = jnp.mean(xc * xc, axis=-1, keepdims=True)
    y = xc * lax.rsqrt(var + 1e-12)
    o_ref[...] = y * gm_ref[...][None, :, :] + bt_ref[...][None, :, :]


def _tc_dense(g3, seg, W2, pos, typ, gamma2, beta2):
    batch, L, wd = g3.shape
    dim = W2.shape[1]
    bb = 256
    grid = (batch // bb,)
    return pl.pallas_call(
        _dense_body,
        grid=grid,
        in_specs=[
            pl.BlockSpec((bb, L, wd), lambda i: (i, 0, 0)),
            pl.BlockSpec((bb, L), lambda i: (i, 0)),
            pl.BlockSpec((wd, dim), lambda i: (0, 0)),
            pl.BlockSpec((L, dim), lambda i: (0, 0)),
            pl.BlockSpec((3, dim), lambda i: (0, 0)),
            pl.BlockSpec((1, dim), lambda i: (0, 0)),
            pl.BlockSpec((1, dim), lambda i: (0, 0)),
        ],
        out_specs=pl.BlockSpec((bb, L, dim), lambda i: (i, 0, 0)),
        out_shape=jax.ShapeDtypeStruct((batch, L, dim), jnp.float32),
    )(g3, seg, W2, pos, typ, gamma2, beta2)


def kernel(input_ids, segment_ids, word_emb, W2, pos_emb, type_emb, gamma, beta):
    batch, L = input_ids.shape
    g3 = jnp.take(word_emb, input_ids, axis=0)
    return _tc_dense(
        g3,
        segment_ids.astype(jnp.int32),
        W2,
        pos_emb,
        type_emb,
        gamma.reshape(1, -1),
        beta.reshape(1, -1),
    )


# speedup vs baseline: 4.2210x; 1.7345x over previous
import functools

import jax
import jax.numpy as jnp
from jax import lax
from jax.experimental import pallas as pl
from jax.experimental.pallas import tpu as pltpu
from jax.experimental.pallas import tpu_sc as plsc

_NW = 32
_CHUNK = 128


def _sc_gather(table, ids_flat):
    ntok = ids_flat.shape[0]
    wd = table.shape[1]
    tok_per_w = ntok // _NW
    n_chunk = tok_per_w // _CHUNK
    mesh = plsc.VectorSubcoreMesh(core_axis_name="c", subcore_axis_name="s")

    @functools.partial(
        pl.kernel,
        mesh=mesh,
        out_type=jax.ShapeDtypeStruct((ntok, wd), jnp.float32),
        scratch_types=[
            pltpu.VMEM((2, _CHUNK), jnp.int32),
            pltpu.VMEM((2, _CHUNK, wd), jnp.float32),
            pltpu.SemaphoreType.DMA,
            pltpu.SemaphoreType.DMA,
        ],
    )
    def k(table_hbm, idx_hbm, out_hbm, idx_v, rows_v, gsem, osem):
        wid = lax.axis_index("s") * 2 + lax.axis_index("c")
        base = wid * tok_per_w

        def fetch(g, slot):
            off = base + g * _CHUNK
            pltpu.sync_copy(idx_hbm.at[pl.ds(off, _CHUNK)], idx_v.at[slot])
            return pltpu.async_copy(table_hbm.at[idx_v.at[slot]],
                                    rows_v.at[slot], gsem)

        fetch(0, 0).wait()

        def body(gg, _):
            for b in range(2):
                g = gg * 2 + b
                nxt = jnp.minimum(g + 1, n_chunk - 1)
                fetch(nxt, 1 - b)
                pltpu.async_copy(
                    rows_v.at[b],
                    out_hbm.at[pl.ds(base + g * _CHUNK, _CHUNK)],
                    osem,
                ).wait()
                pltpu.make_async_copy(table_hbm.at[idx_v.at[1 - b]],
                                      rows_v.at[1 - b], gsem).wait()
            return 0

        lax.fori_loop(0, n_chunk // 2, body, 0)

    return k(table, ids_flat)


def _dense_body(g_ref, s_ref, w_ref, p_ref, t_ref, gm_ref, bt_ref, o_ref):
    bb, L, wd = g_ref.shape
    dim = w_ref.shape[1]
    g = g_ref[...].reshape(bb * L, wd)
    x = jnp.dot(g, w_ref[...], preferred_element_type=jnp.float32)
    x = x.reshape(bb, L, dim)
    x = x + p_ref[...][None, :, :]
    s = s_ref[...]
    t = t_ref[...]
    m1 = lax.broadcast_in_dim((s == 1).astype(jnp.float32), (bb, L, dim), (0, 1))
    m2 = lax.broadcast_in_dim((s == 2).astype(jnp.float32), (bb, L, dim), (0, 1))
    t0 = t[0][None, None, :]
    t1 = t[1][None, None, :]
    t2 = t[2][None, None, :]
    x = x + t0 + m1 * (t1 - t0) + m2 * (t2 - t0)
    mean = jnp.mean(x, axis=-1, keepdims=True)
    xc = x - mean
    var = jnp.mean(xc * xc, axis=-1, keepdims=True)
    y = xc * lax.rsqrt(var + 1e-12)
    o_ref[...] = y * gm_ref[...][None, :, :] + bt_ref[...][None, :, :]


def _tc_dense(g3, seg, W2, pos, typ, gamma2, beta2):
    batch, L, wd = g3.shape
    dim = W2.shape[1]
    bb = 256
    grid = (batch // bb,)
    return pl.pallas_call(
        _dense_body,
        grid=grid,
        in_specs=[
            pl.BlockSpec((bb, L, wd), lambda i: (i, 0, 0)),
            pl.BlockSpec((bb, L), lambda i: (i, 0)),
            pl.BlockSpec((wd, dim), lambda i: (0, 0)),
            pl.BlockSpec((L, dim), lambda i: (0, 0)),
            pl.BlockSpec((3, dim), lambda i: (0, 0)),
            pl.BlockSpec((1, dim), lambda i: (0, 0)),
            pl.BlockSpec((1, dim), lambda i: (0, 0)),
        ],
        out_specs=pl.BlockSpec((bb, L, dim), lambda i: (i, 0, 0)),
        out_shape=jax.ShapeDtypeStruct((batch, L, dim), jnp.float32),
    )(g3, seg, W2, pos, typ, gamma2, beta2)


def kernel(input_ids, segment_ids, word_emb, W2, pos_emb, type_emb, gamma, beta):
    batch, L = input_ids.shape
    ids_flat = input_ids.reshape(-1).astype(jnp.int32)
    g3 = _sc_gather(word_emb, ids_flat).reshape(batch, L, word_emb.shape[1])
    return _tc_dense(
        g3,
        segment_ids.astype(jnp.int32),
        W2,
        pos_emb,
        type_emb,
        gamma.reshape(1, -1),
        beta.reshape(1, -1),
    )


# speedup vs baseline: 4.4195x; 1.0470x over previous
import functools

import jax
import jax.numpy as jnp
from jax import lax
from jax.experimental import pallas as pl
from jax.experimental.pallas import tpu as pltpu
from jax.experimental.pallas import tpu_sc as plsc

_NW = 32
_CHUNK = 128


def _sc_gather(table, ids_flat):
    ntok = ids_flat.shape[0]
    wd = table.shape[1]
    tok_per_w = ntok // _NW
    n_chunk = tok_per_w // _CHUNK
    mesh = plsc.VectorSubcoreMesh(core_axis_name="c", subcore_axis_name="s")

    nb = 4

    @functools.partial(
        pl.kernel,
        mesh=mesh,
        out_type=jax.ShapeDtypeStruct((ntok, wd), jnp.float32),
        scratch_types=[
            pltpu.VMEM((n_chunk, _CHUNK), jnp.int32),
            pltpu.VMEM((nb, _CHUNK, wd), jnp.float32),
            pltpu.SemaphoreType.DMA,
            pltpu.SemaphoreType.DMA,
        ],
    )
    def k(table_hbm, idx_hbm, out_hbm, idx_v, rows_v, gsem, osem):
        wid = lax.axis_index("s") * 2 + lax.axis_index("c")
        base = wid * tok_per_w

        pltpu.sync_copy(idx_hbm.at[pl.ds(wid * n_chunk, n_chunk)], idx_v)

        def gath(g, slot):
            pltpu.async_copy(table_hbm.at[idx_v.at[g]], rows_v.at[slot], gsem)

        def gath_wait(g, slot):
            pltpu.make_async_copy(table_hbm.at[idx_v.at[g]],
                                  rows_v.at[slot], gsem).wait()

        def wr(g, slot):
            pltpu.async_copy(rows_v.at[slot],
                             out_hbm.at[pl.ds(base + g * _CHUNK, _CHUNK)], osem)

        def wr_wait(g, slot):
            pltpu.make_async_copy(
                rows_v.at[slot],
                out_hbm.at[pl.ds(base + g * _CHUNK, _CHUNK)], osem).wait()

        for p in range(nb - 1):
            gath(p, p)

        def body(gg, _):
            for b in range(nb):
                g = gg * nb + b
                gath_wait(g, b)
                wr(g, b)
                @pl.when(g > 0)
                def _():
                    wr_wait(g - 1, (b + nb - 1) % nb)

                @pl.when(g + nb - 1 < n_chunk)
                def _():
                    gath(g + nb - 1, (b + nb - 1) % nb)
            return 0

        lax.fori_loop(0, n_chunk // nb, body, 0)
        wr_wait(n_chunk - 1, nb - 1)

    return k(table, ids_flat.reshape(ntok // _CHUNK, _CHUNK))


def _dense_body(g_ref, s_ref, w_ref, p_ref, t_ref, gm_ref, bt_ref, o_ref):
    bb, L, wd = g_ref.shape
    dim = w_ref.shape[1]
    g = g_ref[...].reshape(bb * L, wd)
    x = jnp.dot(g, w_ref[...], preferred_element_type=jnp.float32)
    x = x.reshape(bb, L, dim)
    x = x + p_ref[...][None, :, :]
    s = s_ref[...]
    t = t_ref[...]
    m1 = lax.broadcast_in_dim((s == 1).astype(jnp.float32), (bb, L, dim), (0, 1))
    m2 = lax.broadcast_in_dim((s == 2).astype(jnp.float32), (bb, L, dim), (0, 1))
    t0 = t[0][None, None, :]
    t1 = t[1][None, None, :]
    t2 = t[2][None, None, :]
    x = x + t0 + m1 * (t1 - t0) + m2 * (t2 - t0)
    mean = jnp.mean(x, axis=-1, keepdims=True)
    xc = x - mean
    var = jnp.mean(xc * xc, axis=-1, keepdims=True)
    y = xc * lax.rsqrt(var + 1e-12)
    o_ref[...] = y * gm_ref[...][None, :, :] + bt_ref[...][None, :, :]


def _tc_dense(g3, seg, W2, pos, typ, gamma2, beta2):
    batch, L, wd = g3.shape
    dim = W2.shape[1]
    bb = 256
    grid = (batch // bb,)
    return pl.pallas_call(
        _dense_body,
        grid=grid,
        in_specs=[
            pl.BlockSpec((bb, L, wd), lambda i: (i, 0, 0)),
            pl.BlockSpec((bb, L), lambda i: (i, 0)),
            pl.BlockSpec((wd, dim), lambda i: (0, 0)),
            pl.BlockSpec((L, dim), lambda i: (0, 0)),
            pl.BlockSpec((3, dim), lambda i: (0, 0)),
            pl.BlockSpec((1, dim), lambda i: (0, 0)),
            pl.BlockSpec((1, dim), lambda i: (0, 0)),
        ],
        out_specs=pl.BlockSpec((bb, L, dim), lambda i: (i, 0, 0)),
        out_shape=jax.ShapeDtypeStruct((batch, L, dim), jnp.float32),
    )(g3, seg, W2, pos, typ, gamma2, beta2)


def kernel(input_ids, segment_ids, word_emb, W2, pos_emb, type_emb, gamma, beta):
    batch, L = input_ids.shape
    ids_flat = input_ids.reshape(-1).astype(jnp.int32)
    g3 = _sc_gather(word_emb, ids_flat).reshape(batch, L, word_emb.shape[1])
    return _tc_dense(
        g3,
        segment_ids.astype(jnp.int32),
        W2,
        pos_emb,
        type_emb,
        gamma.reshape(1, -1),
        beta.reshape(1, -1),
    )


# speedup vs baseline: 5.9131x; 1.3380x over previous
import functools

import jax
import jax.numpy as jnp
from jax import lax
from jax.experimental import pallas as pl
from jax.experimental.pallas import tpu as pltpu
from jax.experimental.pallas import tpu_sc as plsc

_NW = 32
_CHUNK = 128


def _sc_gather(table, ids_flat):
    ntok = ids_flat.shape[0]
    wd = table.shape[1]
    tok_per_w = ntok // _NW
    n_chunk = tok_per_w // _CHUNK
    mesh = plsc.VectorSubcoreMesh(core_axis_name="c", subcore_axis_name="s")

    nb = 4

    @functools.partial(
        pl.kernel,
        mesh=mesh,
        out_type=jax.ShapeDtypeStruct((ntok, wd), jnp.float32),
        scratch_types=[
            pltpu.VMEM((n_chunk, _CHUNK), jnp.int32),
            pltpu.VMEM((nb, _CHUNK, wd), jnp.float32),
            pltpu.SemaphoreType.DMA,
            pltpu.SemaphoreType.DMA,
        ],
    )
    def k(table_hbm, idx_hbm, out_hbm, idx_v, rows_v, gsem, osem):
        wid = lax.axis_index("s") * 2 + lax.axis_index("c")
        base = wid * tok_per_w

        pltpu.sync_copy(idx_hbm.at[pl.ds(wid * n_chunk, n_chunk)], idx_v)

        def gath(g, slot):
            pltpu.async_copy(table_hbm.at[idx_v.at[g]], rows_v.at[slot], gsem)

        def gath_wait(g, slot):
            pltpu.make_async_copy(table_hbm.at[idx_v.at[g]],
                                  rows_v.at[slot], gsem).wait()

        def wr(g, slot):
            pltpu.async_copy(rows_v.at[slot],
                             out_hbm.at[pl.ds(base + g * _CHUNK, _CHUNK)], osem)

        def wr_wait(g, slot):
            pltpu.make_async_copy(
                rows_v.at[slot],
                out_hbm.at[pl.ds(base + g * _CHUNK, _CHUNK)], osem).wait()

        for p in range(nb - 1):
            gath(p, p)

        def body(gg, _):
            for b in range(nb):
                g = gg * nb + b
                gath_wait(g, b)
                wr(g, b)
                @pl.when(g > 0)
                def _():
                    wr_wait(g - 1, (b + nb - 1) % nb)

                @pl.when(g + nb - 1 < n_chunk)
                def _():
                    gath(g + nb - 1, (b + nb - 1) % nb)
            return 0

        lax.fori_loop(0, n_chunk // nb, body, 0)
        wr_wait(n_chunk - 1, nb - 1)

    return k(table, ids_flat.reshape(ntok // _CHUNK, _CHUNK))


def _dense_body(g_ref, oh_ref, w_ref, pt_ref, gm_ref, bt_ref, o_ref):
    bb, L, dim = o_ref.shape
    g = g_ref[...]
    oh = oh_ref[...].astype(jnp.float32)
    x = jnp.dot(g, w_ref[...], preferred_element_type=jnp.float32)
    x = x + jnp.dot(oh, pt_ref[...], preferred_element_type=jnp.float32)
    mean = jnp.sum(x, axis=-1, keepdims=True) * (1.0 / dim)
    xc = x - mean
    var = jnp.sum(xc * xc, axis=-1, keepdims=True) * (1.0 / dim)
    y = xc * lax.rsqrt(var + 1e-12)
    y = y * gm_ref[...] + bt_ref[...]
    o_ref[...] = y.reshape(bb, L, dim)


def _tc_dense(g2, oh2, W2, PTa, gamma2, beta2, batch, L):
    ntok, wd = g2.shape
    dim = W2.shape[1]
    bb = 256
    grid = (batch // bb,)
    return pl.pallas_call(
        _dense_body,
        grid=grid,
        in_specs=[
            pl.BlockSpec((bb * L, wd), lambda i: (i, 0)),
            pl.BlockSpec((bb * L, 64), lambda i: (i, 0)),
            pl.BlockSpec((wd, dim), lambda i: (0, 0)),
            pl.BlockSpec((64, dim), lambda i: (0, 0)),
            pl.BlockSpec((1, dim), lambda i: (0, 0)),
            pl.BlockSpec((1, dim), lambda i: (0, 0)),
        ],
        out_specs=pl.BlockSpec((bb, L, dim), lambda i: (i, 0, 0)),
        out_shape=jax.ShapeDtypeStruct((batch, L, dim), jnp.float32),
    )(g2, oh2, W2, PTa, gamma2, beta2)


def kernel(input_ids, segment_ids, word_emb, W2, pos_emb, type_emb, gamma, beta):
    batch, L = input_ids.shape
    dim = W2.shape[1]
    ids_flat = input_ids.reshape(-1).astype(jnp.int32)
    g2 = _sc_gather(word_emb, ids_flat)

    ptid = jnp.arange(L, dtype=jnp.int32)[None, :] * 3 + segment_ids.astype(jnp.int32)
    oh2 = jax.nn.one_hot(ptid.reshape(-1), 64, dtype=jnp.bfloat16)
    PTa = jnp.zeros((64, dim), jnp.float32)
    PTa = PTa.at[: 3 * L].set(
        (pos_emb[:, None, :] + type_emb[None, :, :]).reshape(3 * L, dim))

    return _tc_dense(g2, oh2, W2, PTa, gamma.reshape(1, -1), beta.reshape(1, -1),
                     batch, L)


# speedup vs baseline: 6.1213x; 1.0352x over previous
import functools

import jax
import jax.numpy as jnp
from jax import lax
from jax.experimental import pallas as pl
from jax.experimental.pallas import tpu as pltpu
from jax.experimental.pallas import tpu_sc as plsc

_NW = 32
_CHUNK = 128


def _sc_gather(table, ids_flat):
    ntok = ids_flat.shape[0]
    wd = table.shape[1]
    dt = table.dtype
    tok_per_w = ntok // _NW
    n_chunk = tok_per_w // _CHUNK
    mesh = plsc.VectorSubcoreMesh(core_axis_name="c", subcore_axis_name="s")

    nb = 4

    @functools.partial(
        pl.kernel,
        mesh=mesh,
        out_type=jax.ShapeDtypeStruct((ntok, wd), dt),
        scratch_types=[
            pltpu.VMEM((n_chunk, _CHUNK), jnp.int32),
            pltpu.VMEM((nb, _CHUNK, wd), dt),
            pltpu.SemaphoreType.DMA,
            pltpu.SemaphoreType.DMA,
        ],
    )
    def k(table_hbm, idx_hbm, out_hbm, idx_v, rows_v, gsem, osem):
        wid = lax.axis_index("s") * 2 + lax.axis_index("c")
        base = wid * tok_per_w

        pltpu.sync_copy(idx_hbm.at[pl.ds(wid * n_chunk, n_chunk)], idx_v)

        def gath(g, slot):
            pltpu.async_copy(table_hbm.at[idx_v.at[g]], rows_v.at[slot], gsem)

        def gath_wait(g, slot):
            pltpu.make_async_copy(table_hbm.at[idx_v.at[g]],
                                  rows_v.at[slot], gsem).wait()

        def wr(g, slot):
            pltpu.async_copy(rows_v.at[slot],
                             out_hbm.at[pl.ds(base + g * _CHUNK, _CHUNK)], osem)

        def wr_wait(g, slot):
            pltpu.make_async_copy(
                rows_v.at[slot],
                out_hbm.at[pl.ds(base + g * _CHUNK, _CHUNK)], osem).wait()

        for p in range(nb - 1):
            gath(p, p)

        def body(gg, _):
            for b in range(nb):
                g = gg * nb + b
                gath_wait(g, b)
                wr(g, b)
                @pl.when(g > 0)
                def _():
                    wr_wait(g - 1, (b + nb - 1) % nb)

                @pl.when(g + nb - 1 < n_chunk)
                def _():
                    gath(g + nb - 1, (b + nb - 1) % nb)
            return 0

        lax.fori_loop(0, n_chunk // nb, body, 0)
        wr_wait(n_chunk - 1, nb - 1)

    return k(table, ids_flat.reshape(ntok // _CHUNK, _CHUNK))


def _dense_body(g_ref, oh_ref, w_ref, pt_ref, gm_ref, bt_ref, o_ref):
    bb, L, dim = o_ref.shape
    g = g_ref[...]
    oh = oh_ref[...].astype(jnp.float32)
    x = jnp.dot(g, w_ref[...], preferred_element_type=jnp.float32)
    x = x + jnp.dot(oh, pt_ref[...], preferred_element_type=jnp.float32)
    mean = jnp.sum(x, axis=-1, keepdims=True) * (1.0 / dim)
    xc = x - mean
    var = jnp.sum(xc * xc, axis=-1, keepdims=True) * (1.0 / dim)
    y = xc * lax.rsqrt(var + 1e-12)
    y = y * gm_ref[...] + bt_ref[...]
    o_ref[...] = y.reshape(bb, L, dim)


def _tc_dense(g2, oh2, W2, PTa, gamma2, beta2, batch, L):
    ntok, wd = g2.shape
    dim = W2.shape[1]
    bb = 256
    grid = (batch // bb,)
    return pl.pallas_call(
        _dense_body,
        grid=grid,
        in_specs=[
            pl.BlockSpec((bb * L, wd), lambda i: (i, 0)),
            pl.BlockSpec((bb * L, 128), lambda i: (i, 0)),
            pl.BlockSpec((wd, dim), lambda i: (0, 0)),
            pl.BlockSpec((128, dim), lambda i: (0, 0)),
            pl.BlockSpec((1, dim), lambda i: (0, 0)),
            pl.BlockSpec((1, dim), lambda i: (0, 0)),
        ],
        out_specs=pl.BlockSpec((bb, L, dim), lambda i: (i, 0, 0)),
        out_shape=jax.ShapeDtypeStruct((batch, L, dim), jnp.float32),
    )(g2, oh2, W2, PTa, gamma2, beta2)


def kernel(input_ids, segment_ids, word_emb, W2, pos_emb, type_emb, gamma, beta):
    batch, L = input_ids.shape
    dim = W2.shape[1]
    ids_flat = input_ids.reshape(-1).astype(jnp.int32)
    g2 = _sc_gather(word_emb, ids_flat)

    ptid = jnp.arange(L, dtype=jnp.int32)[None, :] * 3 + segment_ids.astype(jnp.int32)
    oh2 = jax.nn.one_hot(ptid.reshape(-1), 128, dtype=jnp.int8)
    PTa = jnp.zeros((128, dim), jnp.float32)
    PTa = PTa.at[: 3 * L].set(
        (pos_emb[:, None, :] + type_emb[None, :, :]).reshape(3 * L, dim))

    return _tc_dense(g2, oh2, W2, PTa,
                     gamma.reshape(1, -1), beta.reshape(1, -1), batch, L)
